# baseline (speedup 1.0000x reference)
import functools

import jax
import jax.numpy as jnp
from jax import lax
from jax.experimental import pallas as pl
from jax.experimental.pallas import tpu as pltpu
from jax.experimental.pallas import tpu_sc as plsc

B, L, C, H, W = 2, 16, 32, 64, 64
HW = H * W
NW = 32
PX = HW // NW
NV = PX // 16


def _warp_pscan(flows_r, table):
    mesh = plsc.VectorSubcoreMesh(core_axis_name="c", subcore_axis_name="s")

    @functools.partial(
        pl.kernel,
        mesh=mesh,
        out_type=jax.ShapeDtypeStruct((B, L, HW, C), jnp.float32),
        compiler_params=pltpu.CompilerParams(use_tc_tiling_on_sc=False,
                                             needs_layout_passes=False),
        scratch_types=[
            pltpu.VMEM((B, L, 2, PX), jnp.float32),
            pltpu.VMEM((2, PX), jnp.float32),
            pltpu.VMEM((4, PX), jnp.int32),
            pltpu.VMEM((4, PX), jnp.int32),
            pltpu.VMEM((4 * PX + 16,), jnp.float32),
            pltpu.VMEM((4 * PX + 16,), jnp.float32),
            pltpu.VMEM((4, PX, C), jnp.bfloat16),
            pltpu.VMEM((4, PX, C), jnp.bfloat16),
            pltpu.VMEM((PX, C), jnp.float32),
            pltpu.SemaphoreType.DMA,
            pltpu.SemaphoreType.DMA,
        ],
    )
    def warp_kernel(flows_hbm, table_hbm, out_hbm, cumf, gxy, idx_a, idx_b,
                    w_a, w_b, rows_a, rows_b, acc, sem_a, sem_b):
        idxs = (idx_a, idx_b)
        ws = (w_a, w_b)
        rowss = (rows_a, rows_b)
        sems = (sem_a, sem_b)

        wid = lax.axis_index("s") * 2 + lax.axis_index("c")
        base = wid * PX

        pltpu.sync_copy(flows_hbm.at[:, :, :, pl.ds(base, PX)], cumf)

        def csum_step(l, _):
            for bb in range(B):
                for comp in range(2):
                    for j in range(NV):
                        s = pl.ds(j * 16, 16)
                        cumf[bb, l, comp, s] = (cumf[bb, l, comp, s]
                                                + cumf[bb, l - 1, comp, s])
            return 0
        lax.fori_loop(1, L, csum_step, 0)

        for j in range(NV):
            pi = jnp.arange(16, dtype=jnp.int32) + (base + j * 16)
            pxi = lax.rem(pi, W)
            pyi = lax.div(pi, W)
            s = pl.ds(j * 16, 16)
            gxy[0, s] = (pxi.astype(jnp.float32) + 0.5) * (2.0 / W) - 1.0
            gxy[1, s] = (pyi.astype(jnp.float32) + 0.5) * (2.0 / H) - 1.0

        def compute_and_fire(b, t, k, slot):
            idx4 = idxs[slot]
            w4 = ws[slot]
            for j in range(NV):
                s = pl.ds(j * 16, 16)
                relx = cumf[b, t, 0, s] - cumf[b, k, 0, s]
                rely = cumf[b, t, 1, s] - cumf[b, k, 1, s]
                a = (gxy[0, s] + relx) + 1.0
                r = lax.rem(a, 2.0)
                r = jnp.where(r < 0.0, r + 2.0, r)
                fx = r - 1.0
                ixf = ((fx + 1.0) * float(W) - 1.0) * 0.5
                x0 = ixf.astype(jnp.int32)
                x0 = jnp.where(ixf < 0.0, -1, x0)
                wx1 = ixf - x0.astype(jnp.float32)
                wx0 = 1.0 - wx1
                yv = gxy[1, s] + rely
                iyf = ((yv + 1.0) * float(H) - 1.0) * 0.5
                iyf = jnp.minimum(jnp.maximum(iyf, -4.0), float(H) + 4.0)
                y0 = iyf.astype(jnp.int32)
                y0 = jnp.where(y0.astype(jnp.float32) > iyf, y0 - 1, y0)
                wy1 = iyf - y0.astype(jnp.float32)
                wy0 = 1.0 - wy1
                tb = (b * L + k) * HW
                for q in range(4):
                    dy, dx = q >> 1, q & 1
                    xq = x0 + dx
                    yq = y0 + dy
                    valid = ((xq >= 0) & (xq <= W - 1)
                             & (yq >= 0) & (yq <= H - 1))
                    xqc = jnp.clip(xq, 0, W - 1)
                    yqc = jnp.clip(yq, 0, H - 1)
                    wq = (wx1 if dx else wx0) * (wy1 if dy else wy0)
                    idx4[q, s] = tb + yqc * W + xqc
                    w4[pl.ds(q * PX + j * 16, 16)] = jnp.where(valid, wq, 0.0)
            for q in range(4):
                pltpu.async_copy(table_hbm.at[idx4.at[q]],
                                 rowss[slot].at[q], sems[slot])

        def wait_slot(slot):
            for q in range(4):
                pltpu.make_async_copy(table_hbm.at[idxs[slot].at[q]],
                                      rowss[slot].at[q], sems[slot]).wait()

        def accumulate(slot):
            w4 = ws[slot]
            rows = rowss[slot]

            def acc_p(i, _):
                for u in range(2):
                    p = i * 2 + u
                    a0 = acc[p, pl.ds(0, 16)]
                    a1 = acc[p, pl.ds(16, 16)]
                    for q in range(4):
                        wq = w4[pl.ds(q * PX + p, 16)][0]
                        lo, hi = plsc.unpack(
                            rows[q, p, :], format=plsc.PackFormat.INTERLEAVED)
                        a0 = a0 + wq * lo
                        a1 = a1 + wq * hi
                    acc[p, pl.ds(0, 16)] = a0
                    acc[p, pl.ds(16, 16)] = a1
                return 0
            lax.fori_loop(0, PX // 2, acc_p, 0)

        def b_loop(b, _):
            def t_loop(t, _):
                compute_and_fire(b, t, 0, 0)

                def zero_p(p, _):
                    z = jnp.zeros(16, jnp.float32)
                    acc[p, pl.ds(0, 16)] = z
                    acc[p, pl.ds(16, 16)] = z
                    return 0
                lax.fori_loop(0, PX, zero_p, 0)

                def kk_loop(i, _):
                    k0 = i * 2

                    @pl.when(k0 + 1 <= t)
                    def _():
                        compute_and_fire(b, t, k0 + 1, 1)
                    wait_slot(0)
                    accumulate(0)

                    @pl.when(k0 + 1 <= t)
                    def _():
                        @pl.when(k0 + 2 <= t)
                        def _():
                            compute_and_fire(b, t, k0 + 2, 0)
                        wait_slot(1)
                        accumulate(1)
                    return 0
                lax.fori_loop(0, lax.div(t + 2, 2), kk_loop, 0)

                pltpu.sync_copy(acc, out_hbm.at[b, t, pl.ds(base, PX), :])
                return 0
            lax.fori_loop(0, L, t_loop, 0)
            return 0
        lax.fori_loop(0, B, b_loop, 0)

    return warp_kernel(flows_r, table)


@jax.jit
def kernel(flows, images):
    flows_r = flows.reshape(B, L, 2, HW)
    table = images.transpose(0, 1, 3, 4, 2).reshape(B * L * HW, C)
    table = (table.reshape(-1, 2, C // 2).swapaxes(1, 2)
             .reshape(-1, C).astype(jnp.bfloat16))
    out_cl = _warp_pscan(flows_r, table)
    return out_cl.reshape(B, L, H, W, C).transpose(0, 1, 4, 2, 3)


# speedup vs baseline: 1.1339x; 1.1339x over previous
import functools

import jax
import jax.numpy as jnp
from jax import lax
from jax.experimental import pallas as pl
from jax.experimental.pallas import tpu as pltpu
from jax.experimental.pallas import tpu_sc as plsc

B, L, C, H, W = 2, 16, 32, 64, 64
HW = H * W
NW = 32
PX = HW // NW
NV = PX // 16


def _warp_pscan(flows_r, table):
    mesh = plsc.VectorSubcoreMesh(core_axis_name="c", subcore_axis_name="s")

    @functools.partial(
        pl.kernel,
        mesh=mesh,
        out_type=jax.ShapeDtypeStruct((B, L, HW, C), jnp.float32),
        compiler_params=pltpu.CompilerParams(use_tc_tiling_on_sc=False,
                                             needs_layout_passes=False),
        scratch_types=[
            pltpu.VMEM((B, L, 2, PX), jnp.float32),
            pltpu.VMEM((2, PX), jnp.float32),
            pltpu.VMEM((4, PX), jnp.int32),
            pltpu.VMEM((4, PX), jnp.int32),
            pltpu.VMEM((4 * PX + 16,), jnp.float32),
            pltpu.VMEM((4 * PX + 16,), jnp.float32),
            pltpu.VMEM((4, PX, C), jnp.bfloat16),
            pltpu.VMEM((4, PX, C), jnp.bfloat16),
            pltpu.VMEM((PX, C), jnp.float32),
            pltpu.SemaphoreType.DMA,
            pltpu.SemaphoreType.DMA,
        ],
    )
    def warp_kernel(flows_hbm, table_hbm, out_hbm, cumf, gxy, idx_a, idx_b,
                    w_a, w_b, rows_a, rows_b, acc, sem_a, sem_b):
        idxs = (idx_a, idx_b)
        ws = (w_a, w_b)
        rowss = (rows_a, rows_b)
        sems = (sem_a, sem_b)

        wid = lax.axis_index("s") * 2 + lax.axis_index("c")
        base = wid * PX

        pltpu.sync_copy(flows_hbm.at[:, :, :, pl.ds(base, PX)], cumf)

        def csum_step(l, _):
            for bb in range(B):
                for comp in range(2):
                    for j in range(NV):
                        s = pl.ds(j * 16, 16)
                        cumf[bb, l, comp, s] = (cumf[bb, l, comp, s]
                                                + cumf[bb, l - 1, comp, s])
            return 0
        lax.fori_loop(1, L, csum_step, 0)

        for j in range(NV):
            pi = jnp.arange(16, dtype=jnp.int32) + (base + j * 16)
            pxi = lax.rem(pi, W)
            pyi = lax.div(pi, W)
            s = pl.ds(j * 16, 16)
            gxy[0, s] = (pxi.astype(jnp.float32) + 0.5) * (2.0 / W) - 1.0
            gxy[1, s] = (pyi.astype(jnp.float32) + 0.5) * (2.0 / H) - 1.0

        def compute_and_fire(b, t, k, slot):
            idx4 = idxs[slot]
            w4 = ws[slot]
            for j in range(NV):
                s = pl.ds(j * 16, 16)
                relx = cumf[b, t, 0, s] - cumf[b, k, 0, s]
                rely = cumf[b, t, 1, s] - cumf[b, k, 1, s]
                a = (gxy[0, s] + relx) + 1.0
                r = lax.rem(a, 2.0)
                r = jnp.where(r < 0.0, r + 2.0, r)
                fx = r - 1.0
                ixf = ((fx + 1.0) * float(W) - 1.0) * 0.5
                x0 = ixf.astype(jnp.int32)
                x0 = jnp.where(ixf < 0.0, -1, x0)
                wx1 = ixf - x0.astype(jnp.float32)
                wx0 = 1.0 - wx1
                yv = gxy[1, s] + rely
                iyf = ((yv + 1.0) * float(H) - 1.0) * 0.5
                iyf = jnp.minimum(jnp.maximum(iyf, -4.0), float(H) + 4.0)
                y0 = iyf.astype(jnp.int32)
                y0 = jnp.where(y0.astype(jnp.float32) > iyf, y0 - 1, y0)
                wy1 = iyf - y0.astype(jnp.float32)
                wy0 = 1.0 - wy1
                tb = (b * L + k) * HW
                for q in range(4):
                    dy, dx = q >> 1, q & 1
                    xq = x0 + dx
                    yq = y0 + dy
                    valid = ((xq >= 0) & (xq <= W - 1)
                             & (yq >= 0) & (yq <= H - 1))
                    xqc = jnp.clip(xq, 0, W - 1)
                    yqc = jnp.clip(yq, 0, H - 1)
                    wq = (wx1 if dx else wx0) * (wy1 if dy else wy0)
                    idx4[q, s] = tb + yqc * W + xqc
                    w4[pl.ds(q * PX + j * 16, 16)] = jnp.where(valid, wq, 0.0)
            for q in range(4):
                pltpu.async_copy(table_hbm.at[idx4.at[q]],
                                 rowss[slot].at[q], sems[slot])

        def wait_slot(slot):
            for q in range(4):
                pltpu.make_async_copy(table_hbm.at[idxs[slot].at[q]],
                                      rowss[slot].at[q], sems[slot]).wait()

        def accumulate(slot):
            w4 = ws[slot]
            rows = rowss[slot]

            @plsc.parallel_loop(0, PX, unroll=4)
            def _(p):
                a0 = acc[p, pl.ds(0, 16)]
                a1 = acc[p, pl.ds(16, 16)]
                for q in range(4):
                    wq = w4[pl.ds(q * PX + p, 16)][0]
                    lo, hi = plsc.unpack(
                        rows[q, p, :], format=plsc.PackFormat.INTERLEAVED)
                    a0 = a0 + wq * lo
                    a1 = a1 + wq * hi
                acc[p, pl.ds(0, 16)] = a0
                acc[p, pl.ds(16, 16)] = a1

        def b_loop(b, _):
            def t_loop(t, _):
                compute_and_fire(b, t, 0, 0)

                def zero_p(p, _):
                    z = jnp.zeros(16, jnp.float32)
                    acc[p, pl.ds(0, 16)] = z
                    acc[p, pl.ds(16, 16)] = z
                    return 0
                lax.fori_loop(0, PX, zero_p, 0)

                def kk_loop(i, _):
                    k0 = i * 2

                    @pl.when(k0 + 1 <= t)
                    def _():
                        compute_and_fire(b, t, k0 + 1, 1)
                    wait_slot(0)
                    accumulate(0)

                    @pl.when(k0 + 1 <= t)
                    def _():
                        @pl.when(k0 + 2 <= t)
                        def _():
                            compute_and_fire(b, t, k0 + 2, 0)
                        wait_slot(1)
                        accumulate(1)
                    return 0
                lax.fori_loop(0, lax.div(t + 2, 2), kk_loop, 0)

                pltpu.sync_copy(acc, out_hbm.at[b, t, pl.ds(base, PX), :])
                return 0
            lax.fori_loop(0, L, t_loop, 0)
            return 0
        lax.fori_loop(0, B, b_loop, 0)

    return warp_kernel(flows_r, table)


@jax.jit
def kernel(flows, images):
    flows_r = flows.reshape(B, L, 2, HW)
    table = images.transpose(0, 1, 3, 4, 2).reshape(B * L * HW, C)
    table = (table.reshape(-1, 2, C // 2).swapaxes(1, 2)
             .reshape(-1, C).astype(jnp.bfloat16))
    out_cl = _warp_pscan(flows_r, table)
    return out_cl.reshape(B, L, H, W, C).transpose(0, 1, 4, 2, 3)


# speedup vs baseline: 1.1601x; 1.0230x over previous
import functools

import jax
import jax.numpy as jnp
from jax import lax
from jax.experimental import pallas as pl
from jax.experimental.pallas import tpu as pltpu
from jax.experimental.pallas import tpu_sc as plsc

B, L, C, H, W = 2, 16, 32, 64, 64
HW = H * W
NW = 32
PX = HW // NW
NV = PX // 16


def _warp_pscan(flows_r, table):
    mesh = plsc.VectorSubcoreMesh(core_axis_name="c", subcore_axis_name="s")

    @functools.partial(
        pl.kernel,
        mesh=mesh,
        out_type=jax.ShapeDtypeStruct((B, L, HW, C), jnp.float32),
        compiler_params=pltpu.CompilerParams(use_tc_tiling_on_sc=False,
                                             needs_layout_passes=False),
        scratch_types=[
            pltpu.VMEM((B, L, 2, PX), jnp.float32),
            pltpu.VMEM((2, PX), jnp.float32),
            pltpu.VMEM((2, PX), jnp.int32),
            pltpu.VMEM((2, PX), jnp.int32),
            pltpu.VMEM((4 * PX + 16,), jnp.float32),
            pltpu.VMEM((4 * PX + 16,), jnp.float32),
            pltpu.VMEM((2, PX, 2 * C), jnp.bfloat16),
            pltpu.VMEM((2, PX, 2 * C), jnp.bfloat16),
            pltpu.VMEM((PX, C), jnp.float32),
            pltpu.SemaphoreType.DMA,
            pltpu.SemaphoreType.DMA,
        ],
    )
    def warp_kernel(flows_hbm, table_hbm, out_hbm, cumf, gxy, idx_a, idx_b,
                    w_a, w_b, rows_a, rows_b, acc, sem_a, sem_b):
        idxs = (idx_a, idx_b)
        ws = (w_a, w_b)
        rowss = (rows_a, rows_b)
        sems = (sem_a, sem_b)

        wid = lax.axis_index("s") * 2 + lax.axis_index("c")
        base = wid * PX

        pltpu.sync_copy(flows_hbm.at[:, :, :, pl.ds(base, PX)], cumf)

        def csum_step(l, _):
            for bb in range(B):
                for comp in range(2):
                    for j in range(NV):
                        s = pl.ds(j * 16, 16)
                        cumf[bb, l, comp, s] = (cumf[bb, l, comp, s]
                                                + cumf[bb, l - 1, comp, s])
            return 0
        lax.fori_loop(1, L, csum_step, 0)

        for j in range(NV):
            pi = jnp.arange(16, dtype=jnp.int32) + (base + j * 16)
            pxi = lax.rem(pi, W)
            pyi = lax.div(pi, W)
            s = pl.ds(j * 16, 16)
            gxy[0, s] = (pxi.astype(jnp.float32) + 0.5) * (2.0 / W) - 1.0
            gxy[1, s] = (pyi.astype(jnp.float32) + 0.5) * (2.0 / H) - 1.0

        def compute_and_fire(b, t, k, slot):
            idx4 = idxs[slot]
            w4 = ws[slot]
            for j in range(NV):
                s = pl.ds(j * 16, 16)
                relx = cumf[b, t, 0, s] - cumf[b, k, 0, s]
                rely = cumf[b, t, 1, s] - cumf[b, k, 1, s]
                a = (gxy[0, s] + relx) + 1.0
                r = lax.rem(a, 2.0)
                r = jnp.where(r < 0.0, r + 2.0, r)
                fx = r - 1.0
                ixf = ((fx + 1.0) * float(W) - 1.0) * 0.5
                x0 = ixf.astype(jnp.int32)
                x0 = jnp.where(ixf < 0.0, -1, x0)
                wx1 = ixf - x0.astype(jnp.float32)
                wx0 = 1.0 - wx1
                yv = gxy[1, s] + rely
                iyf = ((yv + 1.0) * float(H) - 1.0) * 0.5
                iyf = jnp.minimum(jnp.maximum(iyf, -4.0), float(H) + 4.0)
                y0 = iyf.astype(jnp.int32)
                y0 = jnp.where(y0.astype(jnp.float32) > iyf, y0 - 1, y0)
                wy1 = iyf - y0.astype(jnp.float32)
                wy0 = 1.0 - wy1
                tb = (b * L + k) * HW
                x0c = jnp.maximum(x0, 0)
                wfx = jnp.where(x0 < 0, wx1, wx0)
                wsx = jnp.where((x0 < 0) | (x0 >= W - 1), 0.0, wx1)
                for tap in range(2):
                    yq = y0 + tap
                    valid_y = (yq >= 0) & (yq <= H - 1)
                    yqc = jnp.clip(yq, 0, H - 1)
                    wyv = jnp.where(valid_y, wy1 if tap else wy0, 0.0)
                    idx4[tap, s] = tb + yqc * W + x0c
                    w4[pl.ds((2 * tap) * PX + j * 16, 16)] = wyv * wfx
                    w4[pl.ds((2 * tap + 1) * PX + j * 16, 16)] = wyv * wsx
            for tap in range(2):
                pltpu.async_copy(table_hbm.at[idx4.at[tap]],
                                 rowss[slot].at[tap], sems[slot])

        def wait_slot(slot):
            for tap in range(2):
                pltpu.make_async_copy(table_hbm.at[idxs[slot].at[tap]],
                                      rowss[slot].at[tap], sems[slot]).wait()

        def accumulate(slot):
            w4 = ws[slot]
            rows = rowss[slot]

            @plsc.parallel_loop(0, PX, unroll=4)
            def _(p):
                a0 = acc[p, pl.ds(0, 16)]
                a1 = acc[p, pl.ds(16, 16)]
                for tap in range(2):
                    wf = w4[pl.ds((2 * tap) * PX + p, 16)][0]
                    wsc = w4[pl.ds((2 * tap + 1) * PX + p, 16)][0]
                    lo, hi = plsc.unpack(
                        rows[tap, p, pl.ds(0, 32)],
                        format=plsc.PackFormat.INTERLEAVED)
                    a0 = a0 + wf * lo
                    a1 = a1 + wf * hi
                    lo2, hi2 = plsc.unpack(
                        rows[tap, p, pl.ds(32, 32)],
                        format=plsc.PackFormat.INTERLEAVED)
                    a0 = a0 + wsc * lo2
                    a1 = a1 + wsc * hi2
                acc[p, pl.ds(0, 16)] = a0
                acc[p, pl.ds(16, 16)] = a1

        def b_loop(b, _):
            def t_loop(t, _):
                compute_and_fire(b, t, 0, 0)

                def zero_p(p, _):
                    z = jnp.zeros(16, jnp.float32)
                    acc[p, pl.ds(0, 16)] = z
                    acc[p, pl.ds(16, 16)] = z
                    return 0
                lax.fori_loop(0, PX, zero_p, 0)

                def kk_loop(i, _):
                    k0 = i * 2

                    @pl.when(k0 + 1 <= t)
                    def _():
                        compute_and_fire(b, t, k0 + 1, 1)
                    wait_slot(0)
                    accumulate(0)

                    @pl.when(k0 + 1 <= t)
                    def _():
                        @pl.when(k0 + 2 <= t)
                        def _():
                            compute_and_fire(b, t, k0 + 2, 0)
                        wait_slot(1)
                        accumulate(1)
                    return 0
                lax.fori_loop(0, lax.div(t + 2, 2), kk_loop, 0)

                pltpu.sync_copy(acc, out_hbm.at[b, t, pl.ds(base, PX), :])
                return 0
            lax.fori_loop(0, L, t_loop, 0)
            return 0
        lax.fori_loop(0, B, b_loop, 0)

    return warp_kernel(flows_r, table)


@jax.jit
def kernel(flows, images):
    flows_r = flows.reshape(B, L, 2, HW)
    table = images.transpose(0, 1, 3, 4, 2).reshape(B * L * HW, C)
    table = (table.reshape(-1, 2, C // 2).swapaxes(1, 2)
             .reshape(-1, C).astype(jnp.bfloat16))
    nxt = jnp.concatenate([table[1:], jnp.zeros((1, C), jnp.bfloat16)], axis=0)
    table = jnp.concatenate([table, nxt], axis=1)
    out_cl = _warp_pscan(flows_r, table)
    return out_cl.reshape(B, L, H, W, C).transpose(0, 1, 4, 2, 3)


# speedup vs baseline: 1.1937x; 1.0290x over previous
import functools

import jax
import jax.numpy as jnp
from jax import lax
from jax.experimental import pallas as pl
from jax.experimental.pallas import tpu as pltpu
from jax.experimental.pallas import tpu_sc as plsc

B, L, C, H, W = 2, 16, 32, 64, 64
HW = H * W
NW = 32
PX = HW // NW
NV = PX // 16


def _warp_pscan(flows_r, table):
    mesh = plsc.VectorSubcoreMesh(core_axis_name="c", subcore_axis_name="s")

    @functools.partial(
        pl.kernel,
        mesh=mesh,
        out_type=jax.ShapeDtypeStruct((B, L, HW, C), jnp.float32),
        compiler_params=pltpu.CompilerParams(use_tc_tiling_on_sc=False,
                                             needs_layout_passes=False),
        scratch_types=[
            pltpu.VMEM((B, L, 2, PX), jnp.float32),
            pltpu.VMEM((2, PX), jnp.float32),
            pltpu.VMEM((2, PX), jnp.int32),
            pltpu.VMEM((2, PX), jnp.int32),
            pltpu.VMEM((4 * PX + 16,), jnp.float32),
            pltpu.VMEM((4 * PX + 16,), jnp.float32),
            pltpu.VMEM((2, PX, 2 * C), jnp.bfloat16),
            pltpu.VMEM((2, PX, 2 * C), jnp.bfloat16),
            pltpu.VMEM((PX, C), jnp.float32),
            pltpu.SemaphoreType.DMA,
            pltpu.SemaphoreType.DMA,
        ],
    )
    def warp_kernel(flows_hbm, table_hbm, out_hbm, cumf, gxy, idx_a, idx_b,
                    w_a, w_b, rows_a, rows_b, acc, sem_a, sem_b):
        idxs = (idx_a, idx_b)
        ws = (w_a, w_b)
        rowss = (rows_a, rows_b)
        sems = (sem_a, sem_b)

        wid = lax.axis_index("s") * 2 + lax.axis_index("c")
        base = wid * PX

        pltpu.sync_copy(flows_hbm.at[:, :, :, pl.ds(base, PX)], cumf)

        def csum_step(l, _):
            for bb in range(B):
                for comp in range(2):
                    for j in range(NV):
                        s = pl.ds(j * 16, 16)
                        cumf[bb, l, comp, s] = (cumf[bb, l, comp, s]
                                                + cumf[bb, l - 1, comp, s])
            return 0
        lax.fori_loop(1, L, csum_step, 0)

        for j in range(NV):
            pi = jnp.arange(16, dtype=jnp.int32) + (base + j * 16)
            pxi = lax.rem(pi, W)
            pyi = lax.div(pi, W)
            s = pl.ds(j * 16, 16)
            gxy[0, s] = (pxi.astype(jnp.float32) + 0.5) * (2.0 / W) - 1.0
            gxy[1, s] = (pyi.astype(jnp.float32) + 0.5) * (2.0 / H) - 1.0

        def compute_and_fire(b, t, k, slot):
            idx4 = idxs[slot]
            w4 = ws[slot]
            tb = (b * L + k) * HW

            @plsc.parallel_loop(0, NV, unroll=2)
            def _(j):
                s = pl.ds(j * 16, 16)
                relx = cumf[b, t, 0, s] - cumf[b, k, 0, s]
                rely = cumf[b, t, 1, s] - cumf[b, k, 1, s]
                a = (gxy[0, s] + relx) + 1.0
                r = lax.rem(a, 2.0)
                r = jnp.where(r < 0.0, r + 2.0, r)
                fx = r - 1.0
                ixf = ((fx + 1.0) * float(W) - 1.0) * 0.5
                x0 = ixf.astype(jnp.int32)
                x0 = jnp.where(ixf < 0.0, -1, x0)
                wx1 = ixf - x0.astype(jnp.float32)
                wx0 = 1.0 - wx1
                yv = gxy[1, s] + rely
                iyf = ((yv + 1.0) * float(H) - 1.0) * 0.5
                iyf = jnp.minimum(jnp.maximum(iyf, -4.0), float(H) + 4.0)
                y0 = iyf.astype(jnp.int32)
                y0 = jnp.where(y0.astype(jnp.float32) > iyf, y0 - 1, y0)
                wy1 = iyf - y0.astype(jnp.float32)
                wy0 = 1.0 - wy1
                x0c = jnp.maximum(x0, 0)
                wfx = jnp.where(x0 < 0, wx1, wx0)
                wsx = jnp.where((x0 < 0) | (x0 >= W - 1), 0.0, wx1)
                for tap in range(2):
                    yq = y0 + tap
                    valid_y = (yq >= 0) & (yq <= H - 1)
                    yqc = jnp.clip(yq, 0, H - 1)
                    wyv = jnp.where(valid_y, wy1 if tap else wy0, 0.0)
                    idx4[tap, s] = tb + yqc * W + x0c
                    w4[pl.ds((2 * tap) * PX + j * 16, 16)] = wyv * wfx
                    w4[pl.ds((2 * tap + 1) * PX + j * 16, 16)] = wyv * wsx
            for tap in range(2):
                pltpu.async_copy(table_hbm.at[idx4.at[tap]],
                                 rowss[slot].at[tap], sems[slot])

        def wait_slot(slot):
            for tap in range(2):
                pltpu.make_async_copy(table_hbm.at[idxs[slot].at[tap]],
                                      rowss[slot].at[tap], sems[slot]).wait()

        def accumulate(slot):
            w4 = ws[slot]
            rows = rowss[slot]

            @plsc.parallel_loop(0, PX, unroll=8)
            def _(p):
                a0 = acc[p, pl.ds(0, 16)]
                a1 = acc[p, pl.ds(16, 16)]
                for tap in range(2):
                    wf = w4[pl.ds((2 * tap) * PX + p, 16)][0]
                    wsc = w4[pl.ds((2 * tap + 1) * PX + p, 16)][0]
                    lo, hi = plsc.unpack(
                        rows[tap, p, pl.ds(0, 32)],
                        format=plsc.PackFormat.INTERLEAVED)
                    a0 = a0 + wf * lo
                    a1 = a1 + wf * hi
                    lo2, hi2 = plsc.unpack(
                        rows[tap, p, pl.ds(32, 32)],
                        format=plsc.PackFormat.INTERLEAVED)
                    a0 = a0 + wsc * lo2
                    a1 = a1 + wsc * hi2
                acc[p, pl.ds(0, 16)] = a0
                acc[p, pl.ds(16, 16)] = a1

        def b_loop(b, _):
            def t_loop(t, _):
                compute_and_fire(b, t, 0, 0)

                def zero_p(p, _):
                    z = jnp.zeros(16, jnp.float32)
                    acc[p, pl.ds(0, 16)] = z
                    acc[p, pl.ds(16, 16)] = z
                    return 0
                lax.fori_loop(0, PX, zero_p, 0)

                def kk_loop(i, _):
                    k0 = i * 2

                    @pl.when(k0 + 1 <= t)
                    def _():
                        compute_and_fire(b, t, k0 + 1, 1)
                    wait_slot(0)
                    accumulate(0)

                    @pl.when(k0 + 1 <= t)
                    def _():
                        @pl.when(k0 + 2 <= t)
                        def _():
                            compute_and_fire(b, t, k0 + 2, 0)
                        wait_slot(1)
                        accumulate(1)
                    return 0
                lax.fori_loop(0, lax.div(t + 2, 2), kk_loop, 0)

                pltpu.sync_copy(acc, out_hbm.at[b, t, pl.ds(base, PX), :])
                return 0
            lax.fori_loop(0, L, t_loop, 0)
            return 0
        lax.fori_loop(0, B, b_loop, 0)

    return warp_kernel(flows_r, table)


@jax.jit
def kernel(flows, images):
    flows_r = flows.reshape(B, L, 2, HW)
    table = images.transpose(0, 1, 3, 4, 2).reshape(B * L * HW, C)
    table = (table.reshape(-1, 2, C // 2).swapaxes(1, 2)
             .reshape(-1, C).astype(jnp.bfloat16))
    nxt = jnp.concatenate([table[1:], jnp.zeros((1, C), jnp.bfloat16)], axis=0)
    table = jnp.concatenate([table, nxt], axis=1)
    out_cl = _warp_pscan(flows_r, table)
    return out_cl.reshape(B, L, H, W, C).transpose(0, 1, 4, 2, 3)


# speedup vs baseline: 1.2678x; 1.0620x over previous
import functools

import jax
import jax.numpy as jnp
from jax import lax
from jax.experimental import pallas as pl
from jax.experimental.pallas import tpu as pltpu
from jax.experimental.pallas import tpu_sc as plsc

B, L, C, H, W = 2, 16, 32, 64, 64
HW = H * W
NW = 32
PX = HW // NW
NV = PX // 16


def _warp_pscan(flows_r, table):
    mesh = plsc.VectorSubcoreMesh(core_axis_name="c", subcore_axis_name="s")

    @functools.partial(
        pl.kernel,
        mesh=mesh,
        out_type=jax.ShapeDtypeStruct((B, L, HW, C), jnp.float32),
        compiler_params=pltpu.CompilerParams(use_tc_tiling_on_sc=False,
                                             needs_layout_passes=False),
        scratch_types=[
            pltpu.VMEM((B, L, 2, PX), jnp.float32),
            pltpu.VMEM((2, PX), jnp.float32),
            pltpu.VMEM((4, PX), jnp.int32),
            pltpu.VMEM((4, PX), jnp.int32),
            pltpu.VMEM((4 * PX + 16,), jnp.float32),
            pltpu.VMEM((4 * PX + 16,), jnp.float32),
            pltpu.VMEM((4, PX, C), jnp.bfloat16),
            pltpu.VMEM((4, PX, C), jnp.bfloat16),
            pltpu.VMEM((PX, C), jnp.float32),
            pltpu.SemaphoreType.DMA,
            pltpu.SemaphoreType.DMA,
        ],
    )
    def warp_kernel(flows_hbm, table_hbm, out_hbm, cumf, gxy, idx_a, idx_b,
                    w_a, w_b, rows_a, rows_b, acc, sem_a, sem_b):
        idxs = (idx_a, idx_b)
        ws = (w_a, w_b)
        rowss = (rows_a, rows_b)
        sems = (sem_a, sem_b)

        wid = lax.axis_index("s") * 2 + lax.axis_index("c")
        base = wid * PX

        pltpu.sync_copy(flows_hbm.at[:, :, :, pl.ds(base, PX)], cumf)

        def csum_step(l, _):
            for bb in range(B):
                for comp in range(2):
                    for j in range(NV):
                        s = pl.ds(j * 16, 16)
                        cumf[bb, l, comp, s] = (cumf[bb, l, comp, s]
                                                + cumf[bb, l - 1, comp, s])
            return 0
        lax.fori_loop(1, L, csum_step, 0)

        for j in range(NV):
            pi = jnp.arange(16, dtype=jnp.int32) + (base + j * 16)
            pxi = lax.rem(pi, W)
            pyi = lax.div(pi, W)
            s = pl.ds(j * 16, 16)
            gxy[0, s] = (pxi.astype(jnp.float32) + 0.5) * (2.0 / W) - 1.0
            gxy[1, s] = (pyi.astype(jnp.float32) + 0.5) * (2.0 / H) - 1.0

        def compute_and_fire(b, t, k, slot):
            idx4 = idxs[slot]
            w4 = ws[slot]
            tb = (b * L + k) * HW

            @plsc.parallel_loop(0, NV, unroll=2)
            def _(j):
                s = pl.ds(j * 16, 16)
                relx = cumf[b, t, 0, s] - cumf[b, k, 0, s]
                rely = cumf[b, t, 1, s] - cumf[b, k, 1, s]
                a = (gxy[0, s] + relx) + 1.0
                r = lax.rem(a, 2.0)
                r = jnp.where(r < 0.0, r + 2.0, r)
                fx = r - 1.0
                ixf = ((fx + 1.0) * float(W) - 1.0) * 0.5
                x0 = ixf.astype(jnp.int32)
                x0 = jnp.where(ixf < 0.0, -1, x0)
                wx1 = ixf - x0.astype(jnp.float32)
                wx0 = 1.0 - wx1
                yv = gxy[1, s] + rely
                iyf = ((yv + 1.0) * float(H) - 1.0) * 0.5
                iyf = jnp.minimum(jnp.maximum(iyf, -4.0), float(H) + 4.0)
                y0 = iyf.astype(jnp.int32)
                y0 = jnp.where(y0.astype(jnp.float32) > iyf, y0 - 1, y0)
                wy1 = iyf - y0.astype(jnp.float32)
                wy0 = 1.0 - wy1
                x0c = jnp.maximum(x0, 0)
                wfx = jnp.where(x0 < 0, wx1, wx0)
                wsx = jnp.where((x0 < 0) | (x0 >= W - 1), 0.0, wx1)
                dxp = jnp.where(x0c >= W - 1, 0, 1)
                for tap in range(2):
                    yq = y0 + tap
                    valid_y = (yq >= 0) & (yq <= H - 1)
                    yqc = jnp.clip(yq, 0, H - 1)
                    wyv = jnp.where(valid_y, wy1 if tap else wy0, 0.0)
                    rbase = tb + yqc * W + x0c
                    idx4[tap, s] = rbase
                    idx4[2 + tap, s] = rbase + dxp
                    w4[pl.ds((2 * tap) * PX + j * 16, 16)] = wyv * wfx
                    w4[pl.ds((2 * tap + 1) * PX + j * 16, 16)] = wyv * wsx
            for q in range(4):
                pltpu.async_copy(table_hbm.at[idx4.at[q]],
                                 rowss[slot].at[q], sems[slot])

        def wait_slot(slot):
            for q in range(4):
                pltpu.make_async_copy(table_hbm.at[idxs[slot].at[q]],
                                      rowss[slot].at[q], sems[slot]).wait()

        def accumulate(slot):
            w4 = ws[slot]
            rows = rowss[slot]

            @plsc.parallel_loop(0, PX, unroll=8)
            def _(p):
                a0 = acc[p, pl.ds(0, 16)]
                a1 = acc[p, pl.ds(16, 16)]
                for tap in range(2):
                    wf = w4[pl.ds((2 * tap) * PX + p, 16)][0]
                    wsc = w4[pl.ds((2 * tap + 1) * PX + p, 16)][0]
                    lo, hi = plsc.unpack(
                        rows[tap, p, :], format=plsc.PackFormat.INTERLEAVED)
                    a0 = a0 + wf * lo
                    a1 = a1 + wf * hi
                    lo2, hi2 = plsc.unpack(
                        rows[2 + tap, p, :],
                        format=plsc.PackFormat.INTERLEAVED)
                    a0 = a0 + wsc * lo2
                    a1 = a1 + wsc * hi2
                acc[p, pl.ds(0, 16)] = a0
                acc[p, pl.ds(16, 16)] = a1

        def b_loop(b, _):
            def t_loop(t, _):
                compute_and_fire(b, t, 0, 0)

                def zero_p(p, _):
                    z = jnp.zeros(16, jnp.float32)
                    acc[p, pl.ds(0, 16)] = z
                    acc[p, pl.ds(16, 16)] = z
                    return 0
                lax.fori_loop(0, PX, zero_p, 0)

                def kk_loop(i, _):
                    k0 = i * 2

                    @pl.when(k0 + 1 <= t)
                    def _():
                        compute_and_fire(b, t, k0 + 1, 1)
                    wait_slot(0)
                    accumulate(0)

                    @pl.when(k0 + 1 <= t)
                    def _():
                        @pl.when(k0 + 2 <= t)
                        def _():
                            compute_and_fire(b, t, k0 + 2, 0)
                        wait_slot(1)
                        accumulate(1)
                    return 0
                lax.fori_loop(0, lax.div(t + 2, 2), kk_loop, 0)

                pltpu.sync_copy(acc, out_hbm.at[b, t, pl.ds(base, PX), :])
                return 0
            lax.fori_loop(0, L, t_loop, 0)
            return 0
        lax.fori_loop(0, B, b_loop, 0)

    return warp_kernel(flows_r, table)


@jax.jit
def kernel(flows, images):
    flows_r = flows.reshape(B, L, 2, HW)
    table = (images.transpose(0, 1, 3, 4, 2).reshape(B * L * HW, C)
             .astype(jnp.bfloat16))
    out_cl = _warp_pscan(flows_r, table)
    out6 = out_cl.reshape(B, L, H, W, 2, C // 2)
    return out6.transpose(0, 1, 5, 4, 2, 3).reshape(B, L, C, H, W)
